# baseline (speedup 1.0000x reference)
import functools

import jax
import jax.numpy as jnp
from jax import lax
from jax.experimental import pallas as pl
from jax.experimental.pallas import tpu as pltpu
from jax.experimental.pallas import tpu_sc as plsc

N = 10000
E = 160000
FIN = 256
HID = 256
C = 64

NP = 10240
EP = 163840
RPT = NP // 16
BM = 512


def _sc_mesh():
    return plsc.VectorSubcoreMesh(core_axis_name="c", subcore_axis_name="s")


def _degree(dstp, zeros_np):
    @functools.partial(
        pl.kernel,
        out_type=jax.ShapeDtypeStruct((32, NP), jnp.float32),
        mesh=_sc_mesh(),
        compiler_params=pltpu.CompilerParams(needs_layout_passes=False),
        scratch_types=[
            pltpu.VMEM((NP,), jnp.float32),
            pltpu.VMEM((EP // 32 // 64, 64), jnp.int32),
        ],
    )
    def k(dst_hbm, z_hbm, out_hbm, dl, didx):
        c = lax.axis_index("c")
        s = lax.axis_index("s")
        wid = s * 2 + c
        nch = EP // 32 // 64
        pltpu.sync_copy(z_hbm, dl)
        pltpu.sync_copy(dst_hbm.at[pl.ds(wid * nch, nch)], didx)
        ones = jnp.ones((16,), jnp.float32)

        def chunk(i, carry):
            t = i // 4
            j = i % 4
            idx = didx[t, pl.ds(16 * j, 16)]
            plsc.addupdate_scatter(dl, [idx], ones)
            return carry

        lax.fori_loop(0, nch * 4, chunk, 0)
        pltpu.sync_copy(dl, out_hbm.at[wid])

    return k(dstp, zeros_np)


def _propagate(gv, src2d, dst2d, zrows, fh, nbuf, echunk, feat_split):
    PNCH = EP // 16 // echunk if feat_split else EP // 32 // echunk

    @functools.partial(
        pl.kernel,
        out_type=jax.ShapeDtypeStruct((2 * NP, fh), jnp.float32),
        mesh=_sc_mesh(),
        compiler_params=pltpu.CompilerParams(
            needs_layout_passes=False, use_tc_tiling_on_sc=(fh % 128 == 0)
        ),
        scratch_types=(
            [pltpu.VMEM((echunk, fh), jnp.float32) for _ in range(nbuf)]
            + [pltpu.VMEM((echunk,), jnp.int32) for _ in range(nbuf)]
            + [pltpu.VMEM((echunk,), jnp.int32) for _ in range(nbuf)]
            + [pltpu.SemaphoreType.DMA for _ in range(nbuf)]
            + [pltpu.SemaphoreType.DMA for _ in range(nbuf)]
            + [pltpu.SemaphoreType.DMA for _ in range(nbuf)]
            + [pltpu.SemaphoreType.DMA for _ in range(nbuf)]
            + [pltpu.VMEM_SHARED((NP, fh), jnp.float32)]
        ),
    )
    def k(g_hbm, src_hbm, dst_hbm, z_hbm, out_hbm, *rest):
        rows = rest[:nbuf]
        sidx = rest[nbuf:2 * nbuf]
        didx = rest[2 * nbuf:3 * nbuf]
        xsem = rest[3 * nbuf:4 * nbuf]
        dsem = rest[4 * nbuf:5 * nbuf]
        gsem = rest[5 * nbuf:6 * nbuf]
        ssem = rest[6 * nbuf:7 * nbuf]
        acc = rest[7 * nbuf]
        c = lax.axis_index("c")
        s = lax.axis_index("s")
        r0 = s * RPT
        pltpu.sync_copy(z_hbm, acc.at[pl.ds(r0, RPT)])
        plsc.subcore_barrier()
        cb = s * PNCH if feat_split else (c * 16 + s) * PNCH

        def xstart(j, b):
            pltpu.make_async_copy(src_hbm.at[cb + j], sidx[b], xsem[b]).start()
            pltpu.make_async_copy(dst_hbm.at[cb + j], didx[b], dsem[b]).start()

        def xwait(j, b):
            pltpu.make_async_copy(src_hbm.at[cb + j], sidx[b], xsem[b]).wait()

        def dwait(j, b):
            pltpu.make_async_copy(dst_hbm.at[cb + j], didx[b], dsem[b]).wait()

        def gstart(j, b):
            if feat_split:
                for j16 in range(echunk // 16):
                    v = sidx[b][pl.ds(16 * j16, 16)]
                    sidx[b][pl.ds(16 * j16, 16)] = v + v + c
            pltpu.make_async_copy(g_hbm.at[sidx[b]], rows[b], gsem[b]).start()

        def gwait(b):
            pltpu.make_async_copy(g_hbm.at[sidx[b]], rows[b], gsem[b]).wait()

        def swait(b):
            pltpu.make_async_copy(rows[b], acc.at[didx[b]], ssem[b]).wait()

        xstart(0, 0)
        xstart(1, 1)
        xwait(0, 0)
        gstart(0, 0)

        def step(t, carry):
            i0 = t * nbuf
            for b in range(nbuf):
                i = i0 + b
                b1 = (b + 1) % nbuf
                b2 = (b + 2) % nbuf

                @pl.when(i + 2 < PNCH)
                def _():
                    @pl.when(i + 2 - nbuf >= 0)
                    def _():
                        swait(b2)

                    xstart(i + 2, b2)

                @pl.when(i + 1 < PNCH)
                def _():
                    xwait(i + 1, b1)
                    gstart(i + 1, b1)

                gwait(b)
                dwait(i, b)
                pltpu.async_copy(rows[b], acc.at[didx[b]], ssem[b], add=True)
            return carry

        lax.fori_loop(0, PNCH // nbuf, step, 0)
        for b in range(nbuf):
            swait(b)
        plsc.subcore_barrier()
        pltpu.sync_copy(
            acc.at[pl.ds(r0, RPT)], out_hbm.at[pl.ds(c * NP + r0, RPT)]
        )

    return k(gv, src2d, dst2d, zrows)


def _matmul(xp, W):
    def body(x_ref, w_ref, o_ref):
        o_ref[...] = jnp.dot(
            x_ref[...], w_ref[...], preferred_element_type=jnp.float32
        )

    return pl.pallas_call(
        body,
        grid=(NP // BM,),
        in_specs=[
            pl.BlockSpec((BM, FIN), lambda i: (i, 0)),
            pl.BlockSpec((FIN, HID), lambda i: (0, 0)),
        ],
        out_specs=pl.BlockSpec((BM, HID), lambda i: (i, 0)),
        out_shape=jax.ShapeDtypeStruct((NP, HID), jnp.float32),
    )(xp, W)


def _scale(parts, h1):
    def body(p_ref, h_ref, g_ref, d_ref):
        deg = (jnp.sum(p_ref[...], axis=0) + 1.0)[:, None]
        d_ref[...] = deg
        g_ref[...] = h_ref[...] * lax.rsqrt(deg)

    return pl.pallas_call(
        body,
        grid=(NP // BM,),
        in_specs=[
            pl.BlockSpec((32, BM), lambda i: (0, i)),
            pl.BlockSpec((BM, HID), lambda i: (i, 0)),
        ],
        out_specs=[
            pl.BlockSpec((BM, HID), lambda i: (i, 0)),
            pl.BlockSpec((BM, 1), lambda i: (i, 0)),
        ],
        out_shape=[
            jax.ShapeDtypeStruct((NP, HID), jnp.float32),
            jax.ShapeDtypeStruct((NP, 1), jnp.float32),
        ],
    )(parts, h1)


def _layer2(acc_a, acc_b, h1, parts, W2, b1):
    def body(pa, pb, h, pr, w, b, z_ref, g_ref):
        dinv = lax.rsqrt(pr[...])
        pre = (
            jnp.concatenate([pa[...], pb[...]], axis=1) * dinv
            + (dinv * dinv) * h[...]
            + b[...]
        )
        h2 = jnp.maximum(pre, 0.0)
        z = jnp.dot(h2, w[...], preferred_element_type=jnp.float32)
        z_ref[...] = z
        g_ref[...] = z * dinv

    return pl.pallas_call(
        body,
        grid=(NP // BM,),
        in_specs=[
            pl.BlockSpec((BM, 128), lambda i: (i, 0)),
            pl.BlockSpec((BM, 128), lambda i: (i + NP // BM, 0)),
            pl.BlockSpec((BM, HID), lambda i: (i, 0)),
            pl.BlockSpec((BM, 1), lambda i: (i, 0)),
            pl.BlockSpec((HID, C), lambda i: (0, 0)),
            pl.BlockSpec((1, HID), lambda i: (0, 0)),
        ],
        out_specs=[
            pl.BlockSpec((BM, C), lambda i: (i, 0)),
            pl.BlockSpec((BM, C), lambda i: (i, 0)),
        ],
        out_shape=[
            jax.ShapeDtypeStruct((NP, C), jnp.float32),
            jax.ShapeDtypeStruct((NP, C), jnp.float32),
        ],
    )(acc_a, acc_b, h1, parts, W2, b1)


def _final(acc_a, acc_b, z, parts, b2):
    def body(pa, pb, zr, pr, b, f_ref, l_ref):
        dinv = lax.rsqrt(pr[...])
        fin = (
            (pa[...] + pb[...]) * dinv
            + (dinv * dinv) * zr[...]
            + b[...]
        )
        m = jnp.max(fin, axis=1, keepdims=True)
        lse = m + jnp.log(jnp.sum(jnp.exp(fin - m), axis=1, keepdims=True))
        f_ref[...] = fin
        l_ref[...] = fin - lse

    BMF = 2000
    return pl.pallas_call(
        body,
        grid=(N // BMF,),
        in_specs=[
            pl.BlockSpec((BMF, C), lambda i: (i, 0)),
            pl.BlockSpec((BMF, C), lambda i: (i, 0)),
            pl.BlockSpec((BMF, C), lambda i: (i, 0)),
            pl.BlockSpec((BMF, 1), lambda i: (i, 0)),
            pl.BlockSpec((1, C), lambda i: (0, 0)),
        ],
        out_specs=[
            pl.BlockSpec((BMF, C), lambda i: (i, 0)),
            pl.BlockSpec((BMF, C), lambda i: (i, 0)),
        ],
        out_shape=[
            jax.ShapeDtypeStruct((N, C), jnp.float32),
            jax.ShapeDtypeStruct((N, C), jnp.float32),
        ],
    )(acc_a, acc_b, z, parts, b2)


def kernel(x, edge_index, W1, b1, W2, b2):
    src = edge_index[0]
    dst = edge_index[1]
    pad = EP - E
    iota = jnp.arange(pad, dtype=jnp.int32)
    srcf = jnp.concatenate([src, iota % N])
    dstf = jnp.concatenate([dst, N + iota % (NP - N)])
    src64 = srcf.reshape(EP // 64, 64)
    dst64 = dstf.reshape(EP // 64, 64)
    src128 = srcf.reshape(EP // 128, 128)
    dst128 = dstf.reshape(EP // 128, 128)
    xp = jnp.pad(x, ((0, NP - N), (0, 0)))

    zeros_np = jnp.zeros((NP,), jnp.float32)
    z1 = jnp.zeros((RPT, 128), jnp.float32)
    z2 = jnp.zeros((RPT, C), jnp.float32)

    parts = _degree(dst64, zeros_np)
    h1 = _matmul(xp, W1)
    g1, deg = _scale(parts, h1)
    P1 = _propagate(g1.reshape(2 * NP, 128), src64, dst64, z1, 128, 4, 64, True)
    z, g2 = _layer2(P1, P1, h1, deg, W2, b1.reshape(1, HID))
    P2 = _propagate(g2, src128, dst128, z2, C, 4, 128, False)
    fin, lsm = _final(P2[:NP], P2[NP:], z, deg, b2.reshape(1, C))
    return fin, lsm
